# baseline (speedup 1.0000x reference)
import jax
import jax.numpy as jnp
from jax import lax
from jax.experimental import pallas as pl
from jax.experimental.pallas import tpu as pltpu
from jax.experimental.pallas import tpu_sc as plsc

N_NODES = 10000
N_FEAT = 128
HIDDEN = 128
N_GRAPHS = 512
N_EDGES = 320000

NC = 2
NS = 16
CHUNK = 128
E_PAD = 327680
ECH = E_PAD // NC // NS // CHUNK
NCHT = E_PAD // CHUNK
PCH0 = NCHT * 3 // 16
PCH1 = NCHT - PCH0
DUMP = N_NODES
TROWS = 10112
NB = 512
NBLK = 20

_mesh = plsc.VectorSubcoreMesh(
    core_axis_name="c", subcore_axis_name="s", num_cores=NC, num_subcores=NS)


def _deg_body(zeros_hbm, ones_hbm, dst_hbm, out_hbm, acc, ones_v, dA, dB,
              isA, isB):
    c = lax.axis_index("c")
    s = lax.axis_index("s")
    rpt = TROWS // NS
    pltpu.sync_copy(zeros_hbm.at[pl.ds(s * rpt, rpt)], acc.at[pl.ds(s * rpt, rpt)])
    pltpu.sync_copy(ones_hbm, ones_v)
    plsc.subcore_barrier()

    pltpu.sync_copy(dst_hbm.at[c, s, 0], dA)
    pltpu.async_copy(dst_hbm.at[c, s, 1], dB, isB)

    def _step(j, dC, isC, dN, isN):
        @pl.when(j + 1 < ECH)
        def _():
            pltpu.make_async_copy(dst_hbm.at[c, s, j + 1], dN, isN).wait()

        pltpu.sync_copy(ones_v, acc.at[dC], add=True)

        @pl.when(j + 2 < ECH)
        def _():
            pltpu.async_copy(dst_hbm.at[c, s, j + 2], dC, isC)

    def body(j, carry):
        @pl.when(j % 2 == 0)
        def _():
            _step(j, dA, isA, dB, isB)

        @pl.when(j % 2 == 1)
        def _():
            _step(j, dB, isB, dA, isA)

        return carry

    lax.fori_loop(0, ECH, body, 0)
    plsc.subcore_barrier()
    pltpu.sync_copy(acc.at[pl.ds(s * rpt, rpt)], out_hbm.at[c, pl.ds(s * rpt, rpt)])


_deg_call = pl.kernel(
    _deg_body,
    out_type=jax.ShapeDtypeStruct((NC, TROWS, 128), jnp.float32),
    mesh=_mesh,
    scratch_types=[
        pltpu.VMEM_SHARED((TROWS, 128), jnp.float32),
        pltpu.VMEM((CHUNK, 128), jnp.float32),
        pltpu.VMEM((CHUNK,), jnp.int32),
        pltpu.VMEM((CHUNK,), jnp.int32),
        pltpu.SemaphoreType.DMA,
        pltpu.SemaphoreType.DMA,
    ],
)


def _prop_body(z_hbm, zeros_hbm, src_hbm, dst_hbm, out_hbm, acc,
               sA, dA, sB, dB, rowsA, rowsB, gsA, gsB, isA, isB):
    c = lax.axis_index("c")
    s = lax.axis_index("s")
    rpt = TROWS // NS
    @pl.when(c == 0)
    def _():
        pltpu.sync_copy(z_hbm.at[pl.ds(s * rpt, rpt)], acc.at[pl.ds(s * rpt, rpt)])

    @pl.when(c != 0)
    def _():
        pltpu.sync_copy(zeros_hbm.at[pl.ds(s * rpt, rpt)],
                        acc.at[pl.ds(s * rpt, rpt)])

    plsc.subcore_barrier()

    n0 = PCH0 // NS
    n1 = PCH1 // NS
    cnt = jnp.where(c == 0, n0, n1)
    base = jnp.where(c == 0, s * n0, PCH0 + s * n1)

    pltpu.sync_copy(src_hbm.at[base], sA)
    pltpu.sync_copy(dst_hbm.at[base], dA)
    pltpu.async_copy(z_hbm.at[sA], rowsA, gsA)
    pltpu.async_copy(src_hbm.at[base + 1], sB, isB)
    pltpu.async_copy(dst_hbm.at[base + 1], dB, isB)

    def _step(j, sC, dC, rowsC, gsC, isC, sN, dN, rowsN, gsN, isN):
        @pl.when(j + 1 < cnt)
        def _():
            pltpu.make_async_copy(src_hbm.at[base + j + 1], sN, isN).wait()
            pltpu.make_async_copy(dst_hbm.at[base + j + 1], dN, isN).wait()
            pltpu.async_copy(z_hbm.at[sN], rowsN, gsN)

        pltpu.make_async_copy(z_hbm.at[sC], rowsC, gsC).wait()
        pltpu.sync_copy(rowsC, acc.at[dC], add=True)

        @pl.when(j + 2 < cnt)
        def _():
            pltpu.async_copy(src_hbm.at[base + j + 2], sC, isC)
            pltpu.async_copy(dst_hbm.at[base + j + 2], dC, isC)

    def body(j, carry):
        @pl.when(j % 2 == 0)
        def _():
            _step(j, sA, dA, rowsA, gsA, isA, sB, dB, rowsB, gsB, isB)

        @pl.when(j % 2 == 1)
        def _():
            _step(j, sB, dB, rowsB, gsB, isB, sA, dA, rowsA, gsA, isA)

        return carry

    lax.fori_loop(0, cnt, body, 0)
    plsc.subcore_barrier()
    pltpu.sync_copy(acc.at[pl.ds(s * rpt, rpt)], out_hbm.at[c, pl.ds(s * rpt, rpt)])


_prop_call = pl.kernel(
    _prop_body,
    out_type=jax.ShapeDtypeStruct((NC, TROWS, 128), jnp.float32),
    mesh=_mesh,
    scratch_types=[
        pltpu.VMEM_SHARED((TROWS, 128), jnp.float32),
        pltpu.VMEM((CHUNK,), jnp.int32),
        pltpu.VMEM((CHUNK,), jnp.int32),
        pltpu.VMEM((CHUNK,), jnp.int32),
        pltpu.VMEM((CHUNK,), jnp.int32),
        pltpu.VMEM((CHUNK, 128), jnp.float32),
        pltpu.VMEM((CHUNK, 128), jnp.float32),
        pltpu.SemaphoreType.DMA,
        pltpu.SemaphoreType.DMA,
        pltpu.SemaphoreType.DMA,
        pltpu.SemaphoreType.DMA,
    ],
)


def _tc1_body(x_ref, w_ref, deg_ref, z_ref, dinv_ref):
    a = deg_ref[...]
    deg = jnp.max(a[0], axis=-1) + jnp.max(a[1], axis=-1) + 1.0
    dv = lax.rsqrt(deg)
    dinv_ref[...] = dv
    z_ref[...] = dv[:, None] * jnp.dot(x_ref[...], w_ref[...],
                                       preferred_element_type=jnp.float32)


def _tc1(x, w1, degp):
    return pl.pallas_call(
        _tc1_body,
        grid=(NBLK,),
        in_specs=[
            pl.BlockSpec((NB, N_FEAT), lambda i: (i, 0)),
            pl.BlockSpec((N_FEAT, HIDDEN), lambda i: (0, 0)),
            pl.BlockSpec((NC, NB, 128), lambda i: (0, i, 0)),
        ],
        out_specs=[
            pl.BlockSpec((NB, HIDDEN), lambda i: (i, 0)),
            pl.BlockSpec((NB,), lambda i: (i,)),
        ],
        out_shape=[
            jax.ShapeDtypeStruct((TROWS, HIDDEN), jnp.float32),
            jax.ShapeDtypeStruct((N_NODES,), jnp.float32),
        ],
    )(x, w1, degp)


def _tc2_body(s_ref, dinv_ref, b_ref, w_ref, z_ref):
    a = s_ref[...]
    f = a[0] + a[1]
    dv = dinv_ref[...][:, None]
    f = jnp.maximum(dv * f + b_ref[...][None, :], 0.0)
    z_ref[...] = dv * jnp.dot(f, w_ref[...], preferred_element_type=jnp.float32)


def _tc2(sp, dinv, b, w):
    return pl.pallas_call(
        _tc2_body,
        grid=(NBLK,),
        in_specs=[
            pl.BlockSpec((NC, NB, HIDDEN), lambda i: (0, i, 0)),
            pl.BlockSpec((NB,), lambda i: (i,)),
            pl.BlockSpec((HIDDEN,), lambda i: (0,)),
            pl.BlockSpec((HIDDEN, HIDDEN), lambda i: (0, 0)),
        ],
        out_specs=pl.BlockSpec((NB, HIDDEN), lambda i: (i, 0)),
        out_shape=jax.ShapeDtypeStruct((TROWS, HIDDEN), jnp.float32),
    )(sp, dinv, b, w)


def _tcf_body(s_ref, dinv_ref, b_ref, batch_ref, wl_ref, bl_ref, wo_ref,
              bo_ref, out_ref, sums_ref, cnt_ref):
    i = pl.program_id(0)
    nblocks = pl.num_programs(0)

    @pl.when(i == 0)
    def _():
        sums_ref[...] = jnp.zeros_like(sums_ref)
        cnt_ref[...] = jnp.zeros_like(cnt_ref)

    a = s_ref[...]
    f = a[0] + a[1]
    dv = dinv_ref[...][:, None]
    f = jnp.maximum(dv * f + b_ref[...][None, :], 0.0)
    rowf = i * NB + lax.broadcasted_iota(jnp.int32, (NB, HIDDEN), 0)
    f = jnp.where(rowf < N_NODES, f, 0.0)

    ids = batch_ref[...][:, None]
    gid = lax.broadcasted_iota(jnp.int32, (NB, N_GRAPHS), 1)
    row = i * NB + lax.broadcasted_iota(jnp.int32, (NB, N_GRAPHS), 0)
    m = ((ids == gid) & (row < N_NODES)).astype(jnp.float32)
    sums_ref[...] += lax.dot_general(m, f, (((0,), (0,)), ((), ())),
                                     preferred_element_type=jnp.float32)
    cnt_ref[...] += jnp.sum(m, axis=0)

    @pl.when(i == nblocks - 1)
    def _():
        pooled = sums_ref[...] / jnp.clip(cnt_ref[...], 1.0, None)[:, None]
        g = jnp.maximum(
            jnp.dot(pooled, wl_ref[...], preferred_element_type=jnp.float32)
            + bl_ref[...][None, :], 0.0)
        out_ref[...] = (jnp.dot(g, wo_ref[...],
                                preferred_element_type=jnp.float32)
                        + bo_ref[...][None, :])


def _tcf(sp, dinv, b, batch, wl, bl, wo, bo):
    return pl.pallas_call(
        _tcf_body,
        grid=(NBLK,),
        in_specs=[
            pl.BlockSpec((NC, NB, HIDDEN), lambda i: (0, i, 0)),
            pl.BlockSpec((NB,), lambda i: (i,)),
            pl.BlockSpec((HIDDEN,), lambda i: (0,)),
            pl.BlockSpec((NB,), lambda i: (i,)),
            pl.BlockSpec((HIDDEN, HIDDEN // 2), lambda i: (0, 0)),
            pl.BlockSpec((HIDDEN // 2,), lambda i: (0,)),
            pl.BlockSpec((HIDDEN // 2, 1), lambda i: (0, 0)),
            pl.BlockSpec((1,), lambda i: (0,)),
        ],
        out_specs=pl.BlockSpec((N_GRAPHS, 1), lambda i: (0, 0)),
        out_shape=jax.ShapeDtypeStruct((N_GRAPHS, 1), jnp.float32),
        scratch_shapes=[
            pltpu.VMEM((N_GRAPHS, HIDDEN), jnp.float32),
            pltpu.VMEM((N_GRAPHS,), jnp.float32),
        ],
    )(sp, dinv, b, batch, wl, bl, wo, bo)


@jax.jit
def _run(x, edge_index, batch, W1, b1, W2, b2, W3, b3, Wl, bl, Wo, bo):
    src = edge_index[0].astype(jnp.int32)
    dst = edge_index[1].astype(jnp.int32)
    pad = E_PAD - N_EDGES
    fill = DUMP + jnp.arange(pad, dtype=jnp.int32) % (TROWS - N_NODES)
    src_p = jnp.concatenate([src, fill])
    dst_p = jnp.concatenate([dst, fill])
    src_c = src_p.reshape(NC, NS, ECH, CHUNK)
    dst_c = dst_p.reshape(NC, NS, ECH, CHUNK)
    src_t = src_p.reshape(NCHT, CHUNK)
    dst_t = dst_p.reshape(NCHT, CHUNK)
    zeros = jnp.zeros((TROWS, 128), jnp.float32)
    ones = jnp.ones((CHUNK, 128), jnp.float32)
    batch_i = batch.astype(jnp.int32)

    degp = _deg_call(zeros, ones, dst_c)
    z1, dinv = _tc1(x, W1, degp)
    s1 = _prop_call(z1, zeros, src_t, dst_t)
    z2 = _tc2(s1, dinv, b1, W2)
    s2 = _prop_call(z2, zeros, src_t, dst_t)
    z3 = _tc2(s2, dinv, b2, W3)
    s3 = _prop_call(z3, zeros, src_t, dst_t)
    return _tcf(s3, dinv, b3, batch_i, Wl, bl, Wo, bo)


def kernel(x, edge_index, batch, W1, b1, W2, b2, W3, b3, Wl, bl, Wo, bo):
    return _run(x, edge_index, batch, W1, b1, W2, b2, W3, b3, Wl, bl, Wo, bo)


# speedup vs baseline: 1.0258x; 1.0258x over previous
import jax
import jax.numpy as jnp
from jax import lax
from jax.experimental import pallas as pl
from jax.experimental.pallas import tpu as pltpu
from jax.experimental.pallas import tpu_sc as plsc

N_NODES = 10000
N_FEAT = 128
HIDDEN = 128
N_GRAPHS = 512
N_EDGES = 320000

NC = 2
NS = 16
CHUNK = 128
E_PAD = 327680
ECH = E_PAD // NC // NS // CHUNK
NCHT = E_PAD // CHUNK
PCH0 = 560
PCH1 = NCHT - PCH0
DUMP = N_NODES
TROWS = 10112
NB = 512
NBLK = 20

_mesh = plsc.VectorSubcoreMesh(
    core_axis_name="c", subcore_axis_name="s", num_cores=NC, num_subcores=NS)


def _deg_body(zeros_hbm, ones_hbm, dst_hbm, out_hbm, acc, ones_v, dA, dB,
              isA, isB):
    c = lax.axis_index("c")
    s = lax.axis_index("s")
    rpt = TROWS // NS
    pltpu.sync_copy(zeros_hbm.at[pl.ds(s * rpt, rpt)], acc.at[pl.ds(s * rpt, rpt)])
    pltpu.sync_copy(ones_hbm, ones_v)
    plsc.subcore_barrier()

    pltpu.sync_copy(dst_hbm.at[c, s, 0], dA)
    pltpu.async_copy(dst_hbm.at[c, s, 1], dB, isB)

    def _step(j, dC, isC, dN, isN):
        @pl.when(j + 1 < ECH)
        def _():
            pltpu.make_async_copy(dst_hbm.at[c, s, j + 1], dN, isN).wait()

        pltpu.sync_copy(ones_v, acc.at[dC], add=True)

        @pl.when(j + 2 < ECH)
        def _():
            pltpu.async_copy(dst_hbm.at[c, s, j + 2], dC, isC)

    def body(j, carry):
        @pl.when(j % 2 == 0)
        def _():
            _step(j, dA, isA, dB, isB)

        @pl.when(j % 2 == 1)
        def _():
            _step(j, dB, isB, dA, isA)

        return carry

    lax.fori_loop(0, ECH, body, 0)
    plsc.subcore_barrier()
    pltpu.sync_copy(acc.at[pl.ds(s * rpt, rpt)], out_hbm.at[c, pl.ds(s * rpt, rpt)])


_deg_call = pl.kernel(
    _deg_body,
    out_type=jax.ShapeDtypeStruct((NC, TROWS, 128), jnp.float32),
    mesh=_mesh,
    scratch_types=[
        pltpu.VMEM_SHARED((TROWS, 128), jnp.float32),
        pltpu.VMEM((CHUNK, 128), jnp.float32),
        pltpu.VMEM((CHUNK,), jnp.int32),
        pltpu.VMEM((CHUNK,), jnp.int32),
        pltpu.SemaphoreType.DMA,
        pltpu.SemaphoreType.DMA,
    ],
)


def _prop_body(z_hbm, zeros_hbm, src_hbm, dst_hbm, out_hbm, acc,
               sA, dA, sB, dB, rowsA, rowsB, gsA, gsB, isA, isB):
    c = lax.axis_index("c")
    s = lax.axis_index("s")
    rpt = TROWS // NS
    @pl.when(c == 0)
    def _():
        pltpu.sync_copy(z_hbm.at[pl.ds(s * rpt, rpt)], acc.at[pl.ds(s * rpt, rpt)])

    @pl.when(c != 0)
    def _():
        pltpu.sync_copy(zeros_hbm.at[pl.ds(s * rpt, rpt)],
                        acc.at[pl.ds(s * rpt, rpt)])

    plsc.subcore_barrier()

    n0 = PCH0 // NS
    n1 = PCH1 // NS
    cnt = jnp.where(c == 0, n0, n1)
    base = jnp.where(c == 0, s * n0, PCH0 + s * n1)

    pltpu.sync_copy(src_hbm.at[base], sA)
    pltpu.sync_copy(dst_hbm.at[base], dA)
    pltpu.async_copy(z_hbm.at[sA], rowsA, gsA)
    pltpu.async_copy(src_hbm.at[base + 1], sB, isB)
    pltpu.async_copy(dst_hbm.at[base + 1], dB, isB)

    def _step(j, sC, dC, rowsC, gsC, isC, sN, dN, rowsN, gsN, isN):
        @pl.when(j + 1 < cnt)
        def _():
            pltpu.make_async_copy(src_hbm.at[base + j + 1], sN, isN).wait()
            pltpu.make_async_copy(dst_hbm.at[base + j + 1], dN, isN).wait()
            pltpu.async_copy(z_hbm.at[sN], rowsN, gsN)

        pltpu.make_async_copy(z_hbm.at[sC], rowsC, gsC).wait()
        pltpu.sync_copy(rowsC, acc.at[dC], add=True)

        @pl.when(j + 2 < cnt)
        def _():
            pltpu.async_copy(src_hbm.at[base + j + 2], sC, isC)
            pltpu.async_copy(dst_hbm.at[base + j + 2], dC, isC)

    def body(j, carry):
        @pl.when(j % 2 == 0)
        def _():
            _step(j, sA, dA, rowsA, gsA, isA, sB, dB, rowsB, gsB, isB)

        @pl.when(j % 2 == 1)
        def _():
            _step(j, sB, dB, rowsB, gsB, isB, sA, dA, rowsA, gsA, isA)

        return carry

    lax.fori_loop(0, cnt, body, 0)
    plsc.subcore_barrier()
    pltpu.sync_copy(acc.at[pl.ds(s * rpt, rpt)], out_hbm.at[c, pl.ds(s * rpt, rpt)])


_prop_call = pl.kernel(
    _prop_body,
    out_type=jax.ShapeDtypeStruct((NC, TROWS, 128), jnp.float32),
    mesh=_mesh,
    scratch_types=[
        pltpu.VMEM_SHARED((TROWS, 128), jnp.float32),
        pltpu.VMEM((CHUNK,), jnp.int32),
        pltpu.VMEM((CHUNK,), jnp.int32),
        pltpu.VMEM((CHUNK,), jnp.int32),
        pltpu.VMEM((CHUNK,), jnp.int32),
        pltpu.VMEM((CHUNK, 128), jnp.float32),
        pltpu.VMEM((CHUNK, 128), jnp.float32),
        pltpu.SemaphoreType.DMA,
        pltpu.SemaphoreType.DMA,
        pltpu.SemaphoreType.DMA,
        pltpu.SemaphoreType.DMA,
    ],
)


def _tc1_body(x_ref, w_ref, deg_ref, z_ref, dinv_ref):
    a = deg_ref[...]
    deg = jnp.max(a[0], axis=-1) + jnp.max(a[1], axis=-1) + 1.0
    dv = lax.rsqrt(deg)
    dinv_ref[...] = dv
    z_ref[...] = dv[:, None] * jnp.dot(x_ref[...], w_ref[...],
                                       preferred_element_type=jnp.float32)


def _tc1(x, w1, degp):
    return pl.pallas_call(
        _tc1_body,
        grid=(NBLK,),
        in_specs=[
            pl.BlockSpec((NB, N_FEAT), lambda i: (i, 0)),
            pl.BlockSpec((N_FEAT, HIDDEN), lambda i: (0, 0)),
            pl.BlockSpec((NC, NB, 128), lambda i: (0, i, 0)),
        ],
        out_specs=[
            pl.BlockSpec((NB, HIDDEN), lambda i: (i, 0)),
            pl.BlockSpec((NB,), lambda i: (i,)),
        ],
        out_shape=[
            jax.ShapeDtypeStruct((TROWS, HIDDEN), jnp.float32),
            jax.ShapeDtypeStruct((N_NODES,), jnp.float32),
        ],
    )(x, w1, degp)


def _tc2_body(s_ref, dinv_ref, b_ref, w_ref, z_ref):
    a = s_ref[...]
    f = a[0] + a[1]
    dv = dinv_ref[...][:, None]
    f = jnp.maximum(dv * f + b_ref[...][None, :], 0.0)
    z_ref[...] = dv * jnp.dot(f, w_ref[...], preferred_element_type=jnp.float32)


def _tc2(sp, dinv, b, w):
    return pl.pallas_call(
        _tc2_body,
        grid=(NBLK,),
        in_specs=[
            pl.BlockSpec((NC, NB, HIDDEN), lambda i: (0, i, 0)),
            pl.BlockSpec((NB,), lambda i: (i,)),
            pl.BlockSpec((HIDDEN,), lambda i: (0,)),
            pl.BlockSpec((HIDDEN, HIDDEN), lambda i: (0, 0)),
        ],
        out_specs=pl.BlockSpec((NB, HIDDEN), lambda i: (i, 0)),
        out_shape=jax.ShapeDtypeStruct((TROWS, HIDDEN), jnp.float32),
    )(sp, dinv, b, w)


def _tcf_body(s_ref, dinv_ref, b_ref, batch_ref, wl_ref, bl_ref, wo_ref,
              bo_ref, out_ref, sums_ref, cnt_ref):
    i = pl.program_id(0)
    nblocks = pl.num_programs(0)

    @pl.when(i == 0)
    def _():
        sums_ref[...] = jnp.zeros_like(sums_ref)
        cnt_ref[...] = jnp.zeros_like(cnt_ref)

    a = s_ref[...]
    f = a[0] + a[1]
    dv = dinv_ref[...][:, None]
    f = jnp.maximum(dv * f + b_ref[...][None, :], 0.0)
    rowf = i * NB + lax.broadcasted_iota(jnp.int32, (NB, HIDDEN), 0)
    f = jnp.where(rowf < N_NODES, f, 0.0)

    ids = batch_ref[...][:, None]
    gid = lax.broadcasted_iota(jnp.int32, (NB, N_GRAPHS), 1)
    row = i * NB + lax.broadcasted_iota(jnp.int32, (NB, N_GRAPHS), 0)
    m = ((ids == gid) & (row < N_NODES)).astype(jnp.float32)
    sums_ref[...] += lax.dot_general(m, f, (((0,), (0,)), ((), ())),
                                     preferred_element_type=jnp.float32)
    cnt_ref[...] += jnp.sum(m, axis=0)

    @pl.when(i == nblocks - 1)
    def _():
        pooled = sums_ref[...] / jnp.clip(cnt_ref[...], 1.0, None)[:, None]
        g = jnp.maximum(
            jnp.dot(pooled, wl_ref[...], preferred_element_type=jnp.float32)
            + bl_ref[...][None, :], 0.0)
        out_ref[...] = (jnp.dot(g, wo_ref[...],
                                preferred_element_type=jnp.float32)
                        + bo_ref[...][None, :])


def _tcf(sp, dinv, b, batch, wl, bl, wo, bo):
    return pl.pallas_call(
        _tcf_body,
        grid=(NBLK,),
        in_specs=[
            pl.BlockSpec((NC, NB, HIDDEN), lambda i: (0, i, 0)),
            pl.BlockSpec((NB,), lambda i: (i,)),
            pl.BlockSpec((HIDDEN,), lambda i: (0,)),
            pl.BlockSpec((NB,), lambda i: (i,)),
            pl.BlockSpec((HIDDEN, HIDDEN // 2), lambda i: (0, 0)),
            pl.BlockSpec((HIDDEN // 2,), lambda i: (0,)),
            pl.BlockSpec((HIDDEN // 2, 1), lambda i: (0, 0)),
            pl.BlockSpec((1,), lambda i: (0,)),
        ],
        out_specs=pl.BlockSpec((N_GRAPHS, 1), lambda i: (0, 0)),
        out_shape=jax.ShapeDtypeStruct((N_GRAPHS, 1), jnp.float32),
        scratch_shapes=[
            pltpu.VMEM((N_GRAPHS, HIDDEN), jnp.float32),
            pltpu.VMEM((N_GRAPHS,), jnp.float32),
        ],
    )(sp, dinv, b, batch, wl, bl, wo, bo)


@jax.jit
def _run(x, edge_index, batch, W1, b1, W2, b2, W3, b3, Wl, bl, Wo, bo):
    src = edge_index[0].astype(jnp.int32)
    dst = edge_index[1].astype(jnp.int32)
    pad = E_PAD - N_EDGES
    fill = DUMP + jnp.arange(pad, dtype=jnp.int32) % (TROWS - N_NODES)
    src_p = jnp.concatenate([src, fill])
    dst_p = jnp.concatenate([dst, fill])
    src_c = src_p.reshape(NC, NS, ECH, CHUNK)
    dst_c = dst_p.reshape(NC, NS, ECH, CHUNK)
    src_t = src_p.reshape(NCHT, CHUNK)
    dst_t = dst_p.reshape(NCHT, CHUNK)
    zeros = jnp.zeros((TROWS, 128), jnp.float32)
    ones = jnp.ones((CHUNK, 128), jnp.float32)
    batch_i = batch.astype(jnp.int32)

    degp = _deg_call(zeros, ones, dst_c)
    z1, dinv = _tc1(x, W1, degp)
    s1 = _prop_call(z1, zeros, src_t, dst_t)
    z2 = _tc2(s1, dinv, b1, W2)
    s2 = _prop_call(z2, zeros, src_t, dst_t)
    z3 = _tc2(s2, dinv, b2, W3)
    s3 = _prop_call(z3, zeros, src_t, dst_t)
    return _tcf(s3, dinv, b3, batch_i, Wl, bl, Wo, bo)


def kernel(x, edge_index, batch, W1, b1, W2, b2, W3, b3, Wl, bl, Wo, bo):
    return _run(x, edge_index, batch, W1, b1, W2, b2, W3, b3, Wl, bl, Wo, bo)


# speedup vs baseline: 1.0801x; 1.0530x over previous
import jax
import jax.numpy as jnp
from jax import lax
from jax.experimental import pallas as pl
from jax.experimental.pallas import tpu as pltpu
from jax.experimental.pallas import tpu_sc as plsc

N_NODES = 10000
N_FEAT = 128
HIDDEN = 128
N_GRAPHS = 512
N_EDGES = 320000

NC = 2
NS = 16
CHUNK = 128
E_PAD = 327680
ECH = E_PAD // NC // NS // CHUNK
NCHT = E_PAD // CHUNK
PCH0 = 720
PCH1 = NCHT - PCH0
DUMP = N_NODES
TROWS = 10112
NB = 512
NBLK = 20

_mesh = plsc.VectorSubcoreMesh(
    core_axis_name="c", subcore_axis_name="s", num_cores=NC, num_subcores=NS)


def _deg_body(zeros_hbm, ones_hbm, dst_hbm, out_hbm, acc, ones_v, dA, dB,
              isA, isB):
    c = lax.axis_index("c")
    s = lax.axis_index("s")
    rpt = TROWS // NS
    pltpu.sync_copy(zeros_hbm.at[pl.ds(s * rpt, rpt)], acc.at[pl.ds(s * rpt, rpt)])
    pltpu.sync_copy(ones_hbm, ones_v)
    plsc.subcore_barrier()

    pltpu.sync_copy(dst_hbm.at[c, s, 0], dA)
    pltpu.async_copy(dst_hbm.at[c, s, 1], dB, isB)

    def _step(j, dC, isC, dN, isN):
        @pl.when(j + 1 < ECH)
        def _():
            pltpu.make_async_copy(dst_hbm.at[c, s, j + 1], dN, isN).wait()

        pltpu.sync_copy(ones_v, acc.at[dC], add=True)

        @pl.when(j + 2 < ECH)
        def _():
            pltpu.async_copy(dst_hbm.at[c, s, j + 2], dC, isC)

    def body(j, carry):
        @pl.when(j % 2 == 0)
        def _():
            _step(j, dA, isA, dB, isB)

        @pl.when(j % 2 == 1)
        def _():
            _step(j, dB, isB, dA, isA)

        return carry

    lax.fori_loop(0, ECH, body, 0)
    plsc.subcore_barrier()
    pltpu.sync_copy(acc.at[pl.ds(s * rpt, rpt)], out_hbm.at[c, pl.ds(s * rpt, rpt)])


_deg_call = pl.kernel(
    _deg_body,
    out_type=jax.ShapeDtypeStruct((NC, TROWS, 128), jnp.float32),
    mesh=_mesh,
    scratch_types=[
        pltpu.VMEM_SHARED((TROWS, 128), jnp.float32),
        pltpu.VMEM((CHUNK, 128), jnp.float32),
        pltpu.VMEM((CHUNK,), jnp.int32),
        pltpu.VMEM((CHUNK,), jnp.int32),
        pltpu.SemaphoreType.DMA,
        pltpu.SemaphoreType.DMA,
    ],
)


def _prop_body(z_hbm, zeros_hbm, src_hbm, dst_hbm, out_hbm, acc,
               sA, dA, sB, dB, rowsA, rowsB, gsA, gsB, isA, isB):
    c = lax.axis_index("c")
    s = lax.axis_index("s")
    rpt = TROWS // NS
    @pl.when(c == 0)
    def _():
        pltpu.sync_copy(z_hbm.at[pl.ds(s * rpt, rpt)], acc.at[pl.ds(s * rpt, rpt)])

    @pl.when(c != 0)
    def _():
        pltpu.sync_copy(zeros_hbm.at[pl.ds(s * rpt, rpt)],
                        acc.at[pl.ds(s * rpt, rpt)])

    plsc.subcore_barrier()

    n0 = PCH0 // NS
    n1 = PCH1 // NS
    cnt = jnp.where(c == 0, n0, n1)
    base = jnp.where(c == 0, s * n0, PCH0 + s * n1)

    pltpu.sync_copy(src_hbm.at[base], sA)
    pltpu.sync_copy(dst_hbm.at[base], dA)
    pltpu.async_copy(z_hbm.at[sA], rowsA, gsA)
    pltpu.async_copy(src_hbm.at[base + 1], sB, isB)
    pltpu.async_copy(dst_hbm.at[base + 1], dB, isB)

    def _step(j, sC, dC, rowsC, gsC, isC, sN, dN, rowsN, gsN, isN):
        @pl.when(j + 1 < cnt)
        def _():
            pltpu.make_async_copy(src_hbm.at[base + j + 1], sN, isN).wait()
            pltpu.make_async_copy(dst_hbm.at[base + j + 1], dN, isN).wait()
            pltpu.async_copy(z_hbm.at[sN], rowsN, gsN)

        pltpu.make_async_copy(z_hbm.at[sC], rowsC, gsC).wait()
        pltpu.sync_copy(rowsC, acc.at[dC], add=True)

        @pl.when(j + 2 < cnt)
        def _():
            pltpu.async_copy(src_hbm.at[base + j + 2], sC, isC)
            pltpu.async_copy(dst_hbm.at[base + j + 2], dC, isC)

    def body(j, carry):
        @pl.when(j % 2 == 0)
        def _():
            _step(j, sA, dA, rowsA, gsA, isA, sB, dB, rowsB, gsB, isB)

        @pl.when(j % 2 == 1)
        def _():
            _step(j, sB, dB, rowsB, gsB, isB, sA, dA, rowsA, gsA, isA)

        return carry

    lax.fori_loop(0, cnt, body, 0)
    plsc.subcore_barrier()
    pltpu.sync_copy(acc.at[pl.ds(s * rpt, rpt)], out_hbm.at[c, pl.ds(s * rpt, rpt)])


_prop_call = pl.kernel(
    _prop_body,
    out_type=jax.ShapeDtypeStruct((NC, TROWS, 128), jnp.float32),
    mesh=_mesh,
    scratch_types=[
        pltpu.VMEM_SHARED((TROWS, 128), jnp.float32),
        pltpu.VMEM((CHUNK,), jnp.int32),
        pltpu.VMEM((CHUNK,), jnp.int32),
        pltpu.VMEM((CHUNK,), jnp.int32),
        pltpu.VMEM((CHUNK,), jnp.int32),
        pltpu.VMEM((CHUNK, 128), jnp.float32),
        pltpu.VMEM((CHUNK, 128), jnp.float32),
        pltpu.SemaphoreType.DMA,
        pltpu.SemaphoreType.DMA,
        pltpu.SemaphoreType.DMA,
        pltpu.SemaphoreType.DMA,
    ],
)


def _tc1_body(x_ref, w_ref, deg_ref, z_ref, dinv_ref):
    a = deg_ref[...]
    deg = jnp.max(a[0], axis=-1) + jnp.max(a[1], axis=-1) + 1.0
    dv = lax.rsqrt(deg)
    dinv_ref[...] = dv
    z_ref[...] = dv[:, None] * jnp.dot(x_ref[...], w_ref[...],
                                       preferred_element_type=jnp.float32)


def _tc1(x, w1, degp):
    return pl.pallas_call(
        _tc1_body,
        grid=(NBLK,),
        in_specs=[
            pl.BlockSpec((NB, N_FEAT), lambda i: (i, 0)),
            pl.BlockSpec((N_FEAT, HIDDEN), lambda i: (0, 0)),
            pl.BlockSpec((NC, NB, 128), lambda i: (0, i, 0)),
        ],
        out_specs=[
            pl.BlockSpec((NB, HIDDEN), lambda i: (i, 0)),
            pl.BlockSpec((NB,), lambda i: (i,)),
        ],
        out_shape=[
            jax.ShapeDtypeStruct((TROWS, HIDDEN), jnp.float32),
            jax.ShapeDtypeStruct((N_NODES,), jnp.float32),
        ],
    )(x, w1, degp)


def _tc2_body(s_ref, dinv_ref, b_ref, w_ref, z_ref):
    a = s_ref[...]
    f = a[0] + a[1]
    dv = dinv_ref[...][:, None]
    f = jnp.maximum(dv * f + b_ref[...][None, :], 0.0)
    z_ref[...] = dv * jnp.dot(f, w_ref[...], preferred_element_type=jnp.float32)


def _tc2(sp, dinv, b, w):
    return pl.pallas_call(
        _tc2_body,
        grid=(NBLK,),
        in_specs=[
            pl.BlockSpec((NC, NB, HIDDEN), lambda i: (0, i, 0)),
            pl.BlockSpec((NB,), lambda i: (i,)),
            pl.BlockSpec((HIDDEN,), lambda i: (0,)),
            pl.BlockSpec((HIDDEN, HIDDEN), lambda i: (0, 0)),
        ],
        out_specs=pl.BlockSpec((NB, HIDDEN), lambda i: (i, 0)),
        out_shape=jax.ShapeDtypeStruct((TROWS, HIDDEN), jnp.float32),
    )(sp, dinv, b, w)


def _tcf_body(s_ref, dinv_ref, b_ref, batch_ref, wl_ref, bl_ref, wo_ref,
              bo_ref, out_ref, sums_ref, cnt_ref):
    i = pl.program_id(0)
    nblocks = pl.num_programs(0)

    @pl.when(i == 0)
    def _():
        sums_ref[...] = jnp.zeros_like(sums_ref)
        cnt_ref[...] = jnp.zeros_like(cnt_ref)

    a = s_ref[...]
    f = a[0] + a[1]
    dv = dinv_ref[...][:, None]
    f = jnp.maximum(dv * f + b_ref[...][None, :], 0.0)
    rowf = i * NB + lax.broadcasted_iota(jnp.int32, (NB, HIDDEN), 0)
    f = jnp.where(rowf < N_NODES, f, 0.0)

    ids = batch_ref[...][:, None]
    gid = lax.broadcasted_iota(jnp.int32, (NB, N_GRAPHS), 1)
    row = i * NB + lax.broadcasted_iota(jnp.int32, (NB, N_GRAPHS), 0)
    m = ((ids == gid) & (row < N_NODES)).astype(jnp.float32)
    sums_ref[...] += lax.dot_general(m, f, (((0,), (0,)), ((), ())),
                                     preferred_element_type=jnp.float32)
    cnt_ref[...] += jnp.sum(m, axis=0)

    @pl.when(i == nblocks - 1)
    def _():
        pooled = sums_ref[...] / jnp.clip(cnt_ref[...], 1.0, None)[:, None]
        g = jnp.maximum(
            jnp.dot(pooled, wl_ref[...], preferred_element_type=jnp.float32)
            + bl_ref[...][None, :], 0.0)
        out_ref[...] = (jnp.dot(g, wo_ref[...],
                                preferred_element_type=jnp.float32)
                        + bo_ref[...][None, :])


def _tcf(sp, dinv, b, batch, wl, bl, wo, bo):
    return pl.pallas_call(
        _tcf_body,
        grid=(NBLK,),
        in_specs=[
            pl.BlockSpec((NC, NB, HIDDEN), lambda i: (0, i, 0)),
            pl.BlockSpec((NB,), lambda i: (i,)),
            pl.BlockSpec((HIDDEN,), lambda i: (0,)),
            pl.BlockSpec((NB,), lambda i: (i,)),
            pl.BlockSpec((HIDDEN, HIDDEN // 2), lambda i: (0, 0)),
            pl.BlockSpec((HIDDEN // 2,), lambda i: (0,)),
            pl.BlockSpec((HIDDEN // 2, 1), lambda i: (0, 0)),
            pl.BlockSpec((1,), lambda i: (0,)),
        ],
        out_specs=pl.BlockSpec((N_GRAPHS, 1), lambda i: (0, 0)),
        out_shape=jax.ShapeDtypeStruct((N_GRAPHS, 1), jnp.float32),
        scratch_shapes=[
            pltpu.VMEM((N_GRAPHS, HIDDEN), jnp.float32),
            pltpu.VMEM((N_GRAPHS,), jnp.float32),
        ],
    )(sp, dinv, b, batch, wl, bl, wo, bo)


@jax.jit
def _run(x, edge_index, batch, W1, b1, W2, b2, W3, b3, Wl, bl, Wo, bo):
    src = edge_index[0].astype(jnp.int32)
    dst = edge_index[1].astype(jnp.int32)
    pad = E_PAD - N_EDGES
    fill = DUMP + jnp.arange(pad, dtype=jnp.int32) % (TROWS - N_NODES)
    src_p = jnp.concatenate([src, fill])
    dst_p = jnp.concatenate([dst, fill])
    src_c = src_p.reshape(NC, NS, ECH, CHUNK)
    dst_c = dst_p.reshape(NC, NS, ECH, CHUNK)
    src_t = src_p.reshape(NCHT, CHUNK)
    dst_t = dst_p.reshape(NCHT, CHUNK)
    zeros = jnp.zeros((TROWS, 128), jnp.float32)
    ones = jnp.ones((CHUNK, 128), jnp.float32)
    batch_i = batch.astype(jnp.int32)

    degp = _deg_call(zeros, ones, dst_c)
    z1, dinv = _tc1(x, W1, degp)
    s1 = _prop_call(z1, zeros, src_t, dst_t)
    z2 = _tc2(s1, dinv, b1, W2)
    s2 = _prop_call(z2, zeros, src_t, dst_t)
    z3 = _tc2(s2, dinv, b2, W3)
    s3 = _prop_call(z3, zeros, src_t, dst_t)
    return _tcf(s3, dinv, b3, batch_i, Wl, bl, Wo, bo)


def kernel(x, edge_index, batch, W1, b1, W2, b2, W3, b3, Wl, bl, Wo, bo):
    return _run(x, edge_index, batch, W1, b1, W2, b2, W3, b3, Wl, bl, Wo, bo)


# speedup vs baseline: 1.1112x; 1.0288x over previous
import jax
import jax.numpy as jnp
from jax import lax
from jax.experimental import pallas as pl
from jax.experimental.pallas import tpu as pltpu
from jax.experimental.pallas import tpu_sc as plsc

N_NODES = 10000
N_FEAT = 128
HIDDEN = 128
N_GRAPHS = 512
N_EDGES = 320000

NC = 2
NS = 16
CHUNK = 128
E_PAD = 327680
ECH = E_PAD // NC // NS // CHUNK
NCHT = E_PAD // CHUNK
PCH0 = 800
PCH1 = NCHT - PCH0
DUMP = N_NODES
TROWS = 10112
NB = 512
NBLK = 20

_mesh = plsc.VectorSubcoreMesh(
    core_axis_name="c", subcore_axis_name="s", num_cores=NC, num_subcores=NS)


def _deg_body(zeros_hbm, ones_hbm, dst_hbm, out_hbm, acc, ones_v, dA, dB,
              isA, isB):
    c = lax.axis_index("c")
    s = lax.axis_index("s")
    rpt = TROWS // NS
    pltpu.sync_copy(zeros_hbm.at[pl.ds(s * rpt, rpt)], acc.at[pl.ds(s * rpt, rpt)])
    pltpu.sync_copy(ones_hbm, ones_v)
    plsc.subcore_barrier()

    pltpu.sync_copy(dst_hbm.at[c, s, 0], dA)
    pltpu.async_copy(dst_hbm.at[c, s, 1], dB, isB)

    def _step(j, dC, isC, dN, isN):
        @pl.when(j + 1 < ECH)
        def _():
            pltpu.make_async_copy(dst_hbm.at[c, s, j + 1], dN, isN).wait()

        pltpu.sync_copy(ones_v, acc.at[dC], add=True)

        @pl.when(j + 2 < ECH)
        def _():
            pltpu.async_copy(dst_hbm.at[c, s, j + 2], dC, isC)

    def body(j, carry):
        @pl.when(j % 2 == 0)
        def _():
            _step(j, dA, isA, dB, isB)

        @pl.when(j % 2 == 1)
        def _():
            _step(j, dB, isB, dA, isA)

        return carry

    lax.fori_loop(0, ECH, body, 0)
    plsc.subcore_barrier()
    pltpu.sync_copy(acc.at[pl.ds(s * rpt, rpt)], out_hbm.at[c, pl.ds(s * rpt, rpt)])


_deg_call = pl.kernel(
    _deg_body,
    out_type=jax.ShapeDtypeStruct((NC, TROWS, 128), jnp.float32),
    mesh=_mesh,
    scratch_types=[
        pltpu.VMEM_SHARED((TROWS, 128), jnp.float32),
        pltpu.VMEM((CHUNK, 128), jnp.float32),
        pltpu.VMEM((CHUNK,), jnp.int32),
        pltpu.VMEM((CHUNK,), jnp.int32),
        pltpu.SemaphoreType.DMA,
        pltpu.SemaphoreType.DMA,
    ],
)


def _prop_body(z_hbm, zeros_hbm, src_hbm, dst_hbm, out_hbm, acc,
               sA, dA, sB, dB, rowsA, rowsB, gsA, gsB, isA, isB):
    c = lax.axis_index("c")
    s = lax.axis_index("s")
    rpt = TROWS // NS
    @pl.when(c == 0)
    def _():
        pltpu.sync_copy(z_hbm.at[pl.ds(s * rpt, rpt)], acc.at[pl.ds(s * rpt, rpt)])

    @pl.when(c != 0)
    def _():
        pltpu.sync_copy(zeros_hbm.at[pl.ds(s * rpt, rpt)],
                        acc.at[pl.ds(s * rpt, rpt)])

    plsc.subcore_barrier()

    n0 = PCH0 // NS
    n1 = PCH1 // NS
    cnt = jnp.where(c == 0, n0, n1)
    base = jnp.where(c == 0, s * n0, PCH0 + s * n1)

    pltpu.sync_copy(src_hbm.at[base], sA)
    pltpu.sync_copy(dst_hbm.at[base], dA)
    pltpu.async_copy(z_hbm.at[sA], rowsA, gsA)
    pltpu.async_copy(src_hbm.at[base + 1], sB, isB)
    pltpu.async_copy(dst_hbm.at[base + 1], dB, isB)

    def _step(j, sC, dC, rowsC, gsC, isC, sN, dN, rowsN, gsN, isN):
        @pl.when(j + 1 < cnt)
        def _():
            pltpu.make_async_copy(src_hbm.at[base + j + 1], sN, isN).wait()
            pltpu.make_async_copy(dst_hbm.at[base + j + 1], dN, isN).wait()
            pltpu.async_copy(z_hbm.at[sN], rowsN, gsN)

        pltpu.make_async_copy(z_hbm.at[sC], rowsC, gsC).wait()
        pltpu.sync_copy(rowsC, acc.at[dC], add=True)

        @pl.when(j + 2 < cnt)
        def _():
            pltpu.async_copy(src_hbm.at[base + j + 2], sC, isC)
            pltpu.async_copy(dst_hbm.at[base + j + 2], dC, isC)

    def body(j, carry):
        @pl.when(j % 2 == 0)
        def _():
            _step(j, sA, dA, rowsA, gsA, isA, sB, dB, rowsB, gsB, isB)

        @pl.when(j % 2 == 1)
        def _():
            _step(j, sB, dB, rowsB, gsB, isB, sA, dA, rowsA, gsA, isA)

        return carry

    lax.fori_loop(0, cnt, body, 0)
    plsc.subcore_barrier()
    pltpu.sync_copy(acc.at[pl.ds(s * rpt, rpt)], out_hbm.at[c, pl.ds(s * rpt, rpt)])


_prop_call = pl.kernel(
    _prop_body,
    out_type=jax.ShapeDtypeStruct((NC, TROWS, 128), jnp.float32),
    mesh=_mesh,
    scratch_types=[
        pltpu.VMEM_SHARED((TROWS, 128), jnp.float32),
        pltpu.VMEM((CHUNK,), jnp.int32),
        pltpu.VMEM((CHUNK,), jnp.int32),
        pltpu.VMEM((CHUNK,), jnp.int32),
        pltpu.VMEM((CHUNK,), jnp.int32),
        pltpu.VMEM((CHUNK, 128), jnp.float32),
        pltpu.VMEM((CHUNK, 128), jnp.float32),
        pltpu.SemaphoreType.DMA,
        pltpu.SemaphoreType.DMA,
        pltpu.SemaphoreType.DMA,
        pltpu.SemaphoreType.DMA,
    ],
)


def _tc1_body(x_ref, w_ref, deg_ref, z_ref, dinv_ref):
    a = deg_ref[...]
    deg = jnp.max(a[0], axis=-1) + jnp.max(a[1], axis=-1) + 1.0
    dv = lax.rsqrt(deg)
    dinv_ref[...] = dv
    z_ref[...] = dv[:, None] * jnp.dot(x_ref[...], w_ref[...],
                                       preferred_element_type=jnp.float32)


def _tc1(x, w1, degp):
    return pl.pallas_call(
        _tc1_body,
        grid=(NBLK,),
        in_specs=[
            pl.BlockSpec((NB, N_FEAT), lambda i: (i, 0)),
            pl.BlockSpec((N_FEAT, HIDDEN), lambda i: (0, 0)),
            pl.BlockSpec((NC, NB, 128), lambda i: (0, i, 0)),
        ],
        out_specs=[
            pl.BlockSpec((NB, HIDDEN), lambda i: (i, 0)),
            pl.BlockSpec((NB,), lambda i: (i,)),
        ],
        out_shape=[
            jax.ShapeDtypeStruct((TROWS, HIDDEN), jnp.float32),
            jax.ShapeDtypeStruct((N_NODES,), jnp.float32),
        ],
    )(x, w1, degp)


def _tc2_body(s_ref, dinv_ref, b_ref, w_ref, z_ref):
    a = s_ref[...]
    f = a[0] + a[1]
    dv = dinv_ref[...][:, None]
    f = jnp.maximum(dv * f + b_ref[...][None, :], 0.0)
    z_ref[...] = dv * jnp.dot(f, w_ref[...], preferred_element_type=jnp.float32)


def _tc2(sp, dinv, b, w):
    return pl.pallas_call(
        _tc2_body,
        grid=(NBLK,),
        in_specs=[
            pl.BlockSpec((NC, NB, HIDDEN), lambda i: (0, i, 0)),
            pl.BlockSpec((NB,), lambda i: (i,)),
            pl.BlockSpec((HIDDEN,), lambda i: (0,)),
            pl.BlockSpec((HIDDEN, HIDDEN), lambda i: (0, 0)),
        ],
        out_specs=pl.BlockSpec((NB, HIDDEN), lambda i: (i, 0)),
        out_shape=jax.ShapeDtypeStruct((TROWS, HIDDEN), jnp.float32),
    )(sp, dinv, b, w)


def _tcf_body(s_ref, dinv_ref, b_ref, batch_ref, wl_ref, bl_ref, wo_ref,
              bo_ref, out_ref, sums_ref, cnt_ref):
    i = pl.program_id(0)
    nblocks = pl.num_programs(0)

    @pl.when(i == 0)
    def _():
        sums_ref[...] = jnp.zeros_like(sums_ref)
        cnt_ref[...] = jnp.zeros_like(cnt_ref)

    a = s_ref[...]
    f = a[0] + a[1]
    dv = dinv_ref[...][:, None]
    f = jnp.maximum(dv * f + b_ref[...][None, :], 0.0)
    rowf = i * NB + lax.broadcasted_iota(jnp.int32, (NB, HIDDEN), 0)
    f = jnp.where(rowf < N_NODES, f, 0.0)

    ids = batch_ref[...][:, None]
    gid = lax.broadcasted_iota(jnp.int32, (NB, N_GRAPHS), 1)
    row = i * NB + lax.broadcasted_iota(jnp.int32, (NB, N_GRAPHS), 0)
    m = ((ids == gid) & (row < N_NODES)).astype(jnp.float32)
    sums_ref[...] += lax.dot_general(m, f, (((0,), (0,)), ((), ())),
                                     preferred_element_type=jnp.float32)
    cnt_ref[...] += jnp.sum(m, axis=0)

    @pl.when(i == nblocks - 1)
    def _():
        pooled = sums_ref[...] / jnp.clip(cnt_ref[...], 1.0, None)[:, None]
        g = jnp.maximum(
            jnp.dot(pooled, wl_ref[...], preferred_element_type=jnp.float32)
            + bl_ref[...][None, :], 0.0)
        out_ref[...] = (jnp.dot(g, wo_ref[...],
                                preferred_element_type=jnp.float32)
                        + bo_ref[...][None, :])


def _tcf(sp, dinv, b, batch, wl, bl, wo, bo):
    return pl.pallas_call(
        _tcf_body,
        grid=(NBLK,),
        in_specs=[
            pl.BlockSpec((NC, NB, HIDDEN), lambda i: (0, i, 0)),
            pl.BlockSpec((NB,), lambda i: (i,)),
            pl.BlockSpec((HIDDEN,), lambda i: (0,)),
            pl.BlockSpec((NB,), lambda i: (i,)),
            pl.BlockSpec((HIDDEN, HIDDEN // 2), lambda i: (0, 0)),
            pl.BlockSpec((HIDDEN // 2,), lambda i: (0,)),
            pl.BlockSpec((HIDDEN // 2, 1), lambda i: (0, 0)),
            pl.BlockSpec((1,), lambda i: (0,)),
        ],
        out_specs=pl.BlockSpec((N_GRAPHS, 1), lambda i: (0, 0)),
        out_shape=jax.ShapeDtypeStruct((N_GRAPHS, 1), jnp.float32),
        scratch_shapes=[
            pltpu.VMEM((N_GRAPHS, HIDDEN), jnp.float32),
            pltpu.VMEM((N_GRAPHS,), jnp.float32),
        ],
    )(sp, dinv, b, batch, wl, bl, wo, bo)


@jax.jit
def _run(x, edge_index, batch, W1, b1, W2, b2, W3, b3, Wl, bl, Wo, bo):
    src = edge_index[0].astype(jnp.int32)
    dst = edge_index[1].astype(jnp.int32)
    pad = E_PAD - N_EDGES
    fill = DUMP + jnp.arange(pad, dtype=jnp.int32) % (TROWS - N_NODES)
    src_p = jnp.concatenate([src, fill])
    dst_p = jnp.concatenate([dst, fill])
    src_c = src_p.reshape(NC, NS, ECH, CHUNK)
    dst_c = dst_p.reshape(NC, NS, ECH, CHUNK)
    src_t = src_p.reshape(NCHT, CHUNK)
    dst_t = dst_p.reshape(NCHT, CHUNK)
    zeros = jnp.zeros((TROWS, 128), jnp.float32)
    ones = jnp.ones((CHUNK, 128), jnp.float32)
    batch_i = batch.astype(jnp.int32)

    degp = _deg_call(zeros, ones, dst_c)
    z1, dinv = _tc1(x, W1, degp)
    s1 = _prop_call(z1, zeros, src_t, dst_t)
    z2 = _tc2(s1, dinv, b1, W2)
    s2 = _prop_call(z2, zeros, src_t, dst_t)
    z3 = _tc2(s2, dinv, b2, W3)
    s3 = _prop_call(z3, zeros, src_t, dst_t)
    return _tcf(s3, dinv, b3, batch_i, Wl, bl, Wo, bo)


def kernel(x, edge_index, batch, W1, b1, W2, b2, W3, b3, Wl, bl, Wo, bo):
    return _run(x, edge_index, batch, W1, b1, W2, b2, W3, b3, Wl, bl, Wo, bo)


# speedup vs baseline: 1.1436x; 1.0292x over previous
import jax
import jax.numpy as jnp
from jax import lax
from jax.experimental import pallas as pl
from jax.experimental.pallas import tpu as pltpu
from jax.experimental.pallas import tpu_sc as plsc

N_NODES = 10000
N_FEAT = 128
HIDDEN = 128
N_GRAPHS = 512
N_EDGES = 320000

NC = 2
NS = 16
CHUNK = 128
E_PAD = 327680
ECH = E_PAD // NC // NS // CHUNK
NCHT = E_PAD // CHUNK
PCH0 = 880
PCH1 = NCHT - PCH0
DUMP = N_NODES
TROWS = 10112
NB = 512
NBLK = 20

_mesh = plsc.VectorSubcoreMesh(
    core_axis_name="c", subcore_axis_name="s", num_cores=NC, num_subcores=NS)


def _deg_body(zeros_hbm, ones_hbm, dst_hbm, out_hbm, acc, ones_v, dA, dB,
              isA, isB):
    c = lax.axis_index("c")
    s = lax.axis_index("s")
    rpt = TROWS // NS
    pltpu.sync_copy(zeros_hbm.at[pl.ds(s * rpt, rpt)], acc.at[pl.ds(s * rpt, rpt)])
    pltpu.sync_copy(ones_hbm, ones_v)
    plsc.subcore_barrier()

    pltpu.sync_copy(dst_hbm.at[c, s, 0], dA)
    pltpu.async_copy(dst_hbm.at[c, s, 1], dB, isB)

    def _step(j, dC, isC, dN, isN):
        @pl.when(j + 1 < ECH)
        def _():
            pltpu.make_async_copy(dst_hbm.at[c, s, j + 1], dN, isN).wait()

        pltpu.sync_copy(ones_v, acc.at[dC], add=True)

        @pl.when(j + 2 < ECH)
        def _():
            pltpu.async_copy(dst_hbm.at[c, s, j + 2], dC, isC)

    def body(j, carry):
        @pl.when(j % 2 == 0)
        def _():
            _step(j, dA, isA, dB, isB)

        @pl.when(j % 2 == 1)
        def _():
            _step(j, dB, isB, dA, isA)

        return carry

    lax.fori_loop(0, ECH, body, 0)
    plsc.subcore_barrier()
    pltpu.sync_copy(acc.at[pl.ds(s * rpt, rpt)], out_hbm.at[c, pl.ds(s * rpt, rpt)])


_deg_call = pl.kernel(
    _deg_body,
    out_type=jax.ShapeDtypeStruct((NC, TROWS, 128), jnp.float32),
    mesh=_mesh,
    scratch_types=[
        pltpu.VMEM_SHARED((TROWS, 128), jnp.float32),
        pltpu.VMEM((CHUNK, 128), jnp.float32),
        pltpu.VMEM((CHUNK,), jnp.int32),
        pltpu.VMEM((CHUNK,), jnp.int32),
        pltpu.SemaphoreType.DMA,
        pltpu.SemaphoreType.DMA,
    ],
)


def _prop_body(z_hbm, zeros_hbm, src_hbm, dst_hbm, out_hbm, acc,
               sA, dA, sB, dB, rowsA, rowsB, gsA, gsB, isA, isB):
    c = lax.axis_index("c")
    s = lax.axis_index("s")
    rpt = TROWS // NS
    @pl.when(c == 0)
    def _():
        pltpu.sync_copy(z_hbm.at[pl.ds(s * rpt, rpt)], acc.at[pl.ds(s * rpt, rpt)])

    @pl.when(c != 0)
    def _():
        pltpu.sync_copy(zeros_hbm.at[pl.ds(s * rpt, rpt)],
                        acc.at[pl.ds(s * rpt, rpt)])

    plsc.subcore_barrier()

    n0 = PCH0 // NS
    n1 = PCH1 // NS
    cnt = jnp.where(c == 0, n0, n1)
    base = jnp.where(c == 0, s * n0, PCH0 + s * n1)

    pltpu.sync_copy(src_hbm.at[base], sA)
    pltpu.sync_copy(dst_hbm.at[base], dA)
    pltpu.async_copy(z_hbm.at[sA], rowsA, gsA)
    pltpu.async_copy(src_hbm.at[base + 1], sB, isB)
    pltpu.async_copy(dst_hbm.at[base + 1], dB, isB)

    def _step(j, sC, dC, rowsC, gsC, isC, sN, dN, rowsN, gsN, isN):
        @pl.when(j + 1 < cnt)
        def _():
            pltpu.make_async_copy(src_hbm.at[base + j + 1], sN, isN).wait()
            pltpu.make_async_copy(dst_hbm.at[base + j + 1], dN, isN).wait()
            pltpu.async_copy(z_hbm.at[sN], rowsN, gsN)

        pltpu.make_async_copy(z_hbm.at[sC], rowsC, gsC).wait()
        pltpu.sync_copy(rowsC, acc.at[dC], add=True)

        @pl.when(j + 2 < cnt)
        def _():
            pltpu.async_copy(src_hbm.at[base + j + 2], sC, isC)
            pltpu.async_copy(dst_hbm.at[base + j + 2], dC, isC)

    def body(j, carry):
        @pl.when(j % 2 == 0)
        def _():
            _step(j, sA, dA, rowsA, gsA, isA, sB, dB, rowsB, gsB, isB)

        @pl.when(j % 2 == 1)
        def _():
            _step(j, sB, dB, rowsB, gsB, isB, sA, dA, rowsA, gsA, isA)

        return carry

    lax.fori_loop(0, cnt, body, 0)
    plsc.subcore_barrier()
    pltpu.sync_copy(acc.at[pl.ds(s * rpt, rpt)], out_hbm.at[c, pl.ds(s * rpt, rpt)])


_prop_call = pl.kernel(
    _prop_body,
    out_type=jax.ShapeDtypeStruct((NC, TROWS, 128), jnp.float32),
    mesh=_mesh,
    scratch_types=[
        pltpu.VMEM_SHARED((TROWS, 128), jnp.float32),
        pltpu.VMEM((CHUNK,), jnp.int32),
        pltpu.VMEM((CHUNK,), jnp.int32),
        pltpu.VMEM((CHUNK,), jnp.int32),
        pltpu.VMEM((CHUNK,), jnp.int32),
        pltpu.VMEM((CHUNK, 128), jnp.float32),
        pltpu.VMEM((CHUNK, 128), jnp.float32),
        pltpu.SemaphoreType.DMA,
        pltpu.SemaphoreType.DMA,
        pltpu.SemaphoreType.DMA,
        pltpu.SemaphoreType.DMA,
    ],
)


def _tc1_body(x_ref, w_ref, deg_ref, z_ref, dinv_ref):
    a = deg_ref[...]
    deg = jnp.max(a[0], axis=-1) + jnp.max(a[1], axis=-1) + 1.0
    dv = lax.rsqrt(deg)
    dinv_ref[...] = dv
    z_ref[...] = dv[:, None] * jnp.dot(x_ref[...], w_ref[...],
                                       preferred_element_type=jnp.float32)


def _tc1(x, w1, degp):
    return pl.pallas_call(
        _tc1_body,
        grid=(NBLK,),
        in_specs=[
            pl.BlockSpec((NB, N_FEAT), lambda i: (i, 0)),
            pl.BlockSpec((N_FEAT, HIDDEN), lambda i: (0, 0)),
            pl.BlockSpec((NC, NB, 128), lambda i: (0, i, 0)),
        ],
        out_specs=[
            pl.BlockSpec((NB, HIDDEN), lambda i: (i, 0)),
            pl.BlockSpec((NB,), lambda i: (i,)),
        ],
        out_shape=[
            jax.ShapeDtypeStruct((TROWS, HIDDEN), jnp.float32),
            jax.ShapeDtypeStruct((N_NODES,), jnp.float32),
        ],
    )(x, w1, degp)


def _tc2_body(s_ref, dinv_ref, b_ref, w_ref, z_ref):
    a = s_ref[...]
    f = a[0] + a[1]
    dv = dinv_ref[...][:, None]
    f = jnp.maximum(dv * f + b_ref[...][None, :], 0.0)
    z_ref[...] = dv * jnp.dot(f, w_ref[...], preferred_element_type=jnp.float32)


def _tc2(sp, dinv, b, w):
    return pl.pallas_call(
        _tc2_body,
        grid=(NBLK,),
        in_specs=[
            pl.BlockSpec((NC, NB, HIDDEN), lambda i: (0, i, 0)),
            pl.BlockSpec((NB,), lambda i: (i,)),
            pl.BlockSpec((HIDDEN,), lambda i: (0,)),
            pl.BlockSpec((HIDDEN, HIDDEN), lambda i: (0, 0)),
        ],
        out_specs=pl.BlockSpec((NB, HIDDEN), lambda i: (i, 0)),
        out_shape=jax.ShapeDtypeStruct((TROWS, HIDDEN), jnp.float32),
    )(sp, dinv, b, w)


def _tcf_body(s_ref, dinv_ref, b_ref, batch_ref, wl_ref, bl_ref, wo_ref,
              bo_ref, out_ref, sums_ref, cnt_ref):
    i = pl.program_id(0)
    nblocks = pl.num_programs(0)

    @pl.when(i == 0)
    def _():
        sums_ref[...] = jnp.zeros_like(sums_ref)
        cnt_ref[...] = jnp.zeros_like(cnt_ref)

    a = s_ref[...]
    f = a[0] + a[1]
    dv = dinv_ref[...][:, None]
    f = jnp.maximum(dv * f + b_ref[...][None, :], 0.0)
    rowf = i * NB + lax.broadcasted_iota(jnp.int32, (NB, HIDDEN), 0)
    f = jnp.where(rowf < N_NODES, f, 0.0)

    ids = batch_ref[...][:, None]
    gid = lax.broadcasted_iota(jnp.int32, (NB, N_GRAPHS), 1)
    row = i * NB + lax.broadcasted_iota(jnp.int32, (NB, N_GRAPHS), 0)
    m = ((ids == gid) & (row < N_NODES)).astype(jnp.float32)
    sums_ref[...] += lax.dot_general(m, f, (((0,), (0,)), ((), ())),
                                     preferred_element_type=jnp.float32)
    cnt_ref[...] += jnp.sum(m, axis=0)

    @pl.when(i == nblocks - 1)
    def _():
        pooled = sums_ref[...] / jnp.clip(cnt_ref[...], 1.0, None)[:, None]
        g = jnp.maximum(
            jnp.dot(pooled, wl_ref[...], preferred_element_type=jnp.float32)
            + bl_ref[...][None, :], 0.0)
        out_ref[...] = (jnp.dot(g, wo_ref[...],
                                preferred_element_type=jnp.float32)
                        + bo_ref[...][None, :])


def _tcf(sp, dinv, b, batch, wl, bl, wo, bo):
    return pl.pallas_call(
        _tcf_body,
        grid=(NBLK,),
        in_specs=[
            pl.BlockSpec((NC, NB, HIDDEN), lambda i: (0, i, 0)),
            pl.BlockSpec((NB,), lambda i: (i,)),
            pl.BlockSpec((HIDDEN,), lambda i: (0,)),
            pl.BlockSpec((NB,), lambda i: (i,)),
            pl.BlockSpec((HIDDEN, HIDDEN // 2), lambda i: (0, 0)),
            pl.BlockSpec((HIDDEN // 2,), lambda i: (0,)),
            pl.BlockSpec((HIDDEN // 2, 1), lambda i: (0, 0)),
            pl.BlockSpec((1,), lambda i: (0,)),
        ],
        out_specs=pl.BlockSpec((N_GRAPHS, 1), lambda i: (0, 0)),
        out_shape=jax.ShapeDtypeStruct((N_GRAPHS, 1), jnp.float32),
        scratch_shapes=[
            pltpu.VMEM((N_GRAPHS, HIDDEN), jnp.float32),
            pltpu.VMEM((N_GRAPHS,), jnp.float32),
        ],
    )(sp, dinv, b, batch, wl, bl, wo, bo)


@jax.jit
def _run(x, edge_index, batch, W1, b1, W2, b2, W3, b3, Wl, bl, Wo, bo):
    src = edge_index[0].astype(jnp.int32)
    dst = edge_index[1].astype(jnp.int32)
    pad = E_PAD - N_EDGES
    fill = DUMP + jnp.arange(pad, dtype=jnp.int32) % (TROWS - N_NODES)
    src_p = jnp.concatenate([src, fill])
    dst_p = jnp.concatenate([dst, fill])
    src_c = src_p.reshape(NC, NS, ECH, CHUNK)
    dst_c = dst_p.reshape(NC, NS, ECH, CHUNK)
    src_t = src_p.reshape(NCHT, CHUNK)
    dst_t = dst_p.reshape(NCHT, CHUNK)
    zeros = jnp.zeros((TROWS, 128), jnp.float32)
    ones = jnp.ones((CHUNK, 128), jnp.float32)
    batch_i = batch.astype(jnp.int32)

    degp = _deg_call(zeros, ones, dst_c)
    z1, dinv = _tc1(x, W1, degp)
    s1 = _prop_call(z1, zeros, src_t, dst_t)
    z2 = _tc2(s1, dinv, b1, W2)
    s2 = _prop_call(z2, zeros, src_t, dst_t)
    z3 = _tc2(s2, dinv, b2, W3)
    s3 = _prop_call(z3, zeros, src_t, dst_t)
    return _tcf(s3, dinv, b3, batch_i, Wl, bl, Wo, bo)


def kernel(x, edge_index, batch, W1, b1, W2, b2, W3, b3, Wl, bl, Wo, bo):
    return _run(x, edge_index, batch, W1, b1, W2, b2, W3, b3, Wl, bl, Wo, bo)


# speedup vs baseline: 1.1783x; 1.0303x over previous
import jax
import jax.numpy as jnp
from jax import lax
from jax.experimental import pallas as pl
from jax.experimental.pallas import tpu as pltpu
from jax.experimental.pallas import tpu_sc as plsc

N_NODES = 10000
N_FEAT = 128
HIDDEN = 128
N_GRAPHS = 512
N_EDGES = 320000

NC = 2
NS = 16
CHUNK = 128
E_PAD = 327680
ECH = E_PAD // NC // NS // CHUNK
NCHT = E_PAD // CHUNK
PCH0 = 960
PCH1 = NCHT - PCH0
DUMP = N_NODES
TROWS = 10112
NB = 512
NBLK = 20

_mesh = plsc.VectorSubcoreMesh(
    core_axis_name="c", subcore_axis_name="s", num_cores=NC, num_subcores=NS)


def _deg_body(zeros_hbm, ones_hbm, dst_hbm, out_hbm, acc, ones_v, dA, dB,
              isA, isB):
    c = lax.axis_index("c")
    s = lax.axis_index("s")
    rpt = TROWS // NS
    pltpu.sync_copy(zeros_hbm.at[pl.ds(s * rpt, rpt)], acc.at[pl.ds(s * rpt, rpt)])
    pltpu.sync_copy(ones_hbm, ones_v)
    plsc.subcore_barrier()

    pltpu.sync_copy(dst_hbm.at[c, s, 0], dA)
    pltpu.async_copy(dst_hbm.at[c, s, 1], dB, isB)

    def _step(j, dC, isC, dN, isN):
        @pl.when(j + 1 < ECH)
        def _():
            pltpu.make_async_copy(dst_hbm.at[c, s, j + 1], dN, isN).wait()

        pltpu.sync_copy(ones_v, acc.at[dC], add=True)

        @pl.when(j + 2 < ECH)
        def _():
            pltpu.async_copy(dst_hbm.at[c, s, j + 2], dC, isC)

    def body(j, carry):
        @pl.when(j % 2 == 0)
        def _():
            _step(j, dA, isA, dB, isB)

        @pl.when(j % 2 == 1)
        def _():
            _step(j, dB, isB, dA, isA)

        return carry

    lax.fori_loop(0, ECH, body, 0)
    plsc.subcore_barrier()
    pltpu.sync_copy(acc.at[pl.ds(s * rpt, rpt)], out_hbm.at[c, pl.ds(s * rpt, rpt)])


_deg_call = pl.kernel(
    _deg_body,
    out_type=jax.ShapeDtypeStruct((NC, TROWS, 128), jnp.float32),
    mesh=_mesh,
    scratch_types=[
        pltpu.VMEM_SHARED((TROWS, 128), jnp.float32),
        pltpu.VMEM((CHUNK, 128), jnp.float32),
        pltpu.VMEM((CHUNK,), jnp.int32),
        pltpu.VMEM((CHUNK,), jnp.int32),
        pltpu.SemaphoreType.DMA,
        pltpu.SemaphoreType.DMA,
    ],
)


def _prop_body(z_hbm, zeros_hbm, src_hbm, dst_hbm, out_hbm, acc,
               sA, dA, sB, dB, rowsA, rowsB, gsA, gsB, isA, isB):
    c = lax.axis_index("c")
    s = lax.axis_index("s")
    rpt = TROWS // NS
    @pl.when(c == 0)
    def _():
        pltpu.sync_copy(z_hbm.at[pl.ds(s * rpt, rpt)], acc.at[pl.ds(s * rpt, rpt)])

    @pl.when(c != 0)
    def _():
        pltpu.sync_copy(zeros_hbm.at[pl.ds(s * rpt, rpt)],
                        acc.at[pl.ds(s * rpt, rpt)])

    plsc.subcore_barrier()

    n0 = PCH0 // NS
    n1 = PCH1 // NS
    cnt = jnp.where(c == 0, n0, n1)
    base = jnp.where(c == 0, s * n0, PCH0 + s * n1)

    pltpu.sync_copy(src_hbm.at[base], sA)
    pltpu.sync_copy(dst_hbm.at[base], dA)
    pltpu.async_copy(z_hbm.at[sA], rowsA, gsA)
    pltpu.async_copy(src_hbm.at[base + 1], sB, isB)
    pltpu.async_copy(dst_hbm.at[base + 1], dB, isB)

    def _step(j, sC, dC, rowsC, gsC, isC, sN, dN, rowsN, gsN, isN):
        @pl.when(j + 1 < cnt)
        def _():
            pltpu.make_async_copy(src_hbm.at[base + j + 1], sN, isN).wait()
            pltpu.make_async_copy(dst_hbm.at[base + j + 1], dN, isN).wait()
            pltpu.async_copy(z_hbm.at[sN], rowsN, gsN)

        pltpu.make_async_copy(z_hbm.at[sC], rowsC, gsC).wait()
        pltpu.sync_copy(rowsC, acc.at[dC], add=True)

        @pl.when(j + 2 < cnt)
        def _():
            pltpu.async_copy(src_hbm.at[base + j + 2], sC, isC)
            pltpu.async_copy(dst_hbm.at[base + j + 2], dC, isC)

    def body(j, carry):
        @pl.when(j % 2 == 0)
        def _():
            _step(j, sA, dA, rowsA, gsA, isA, sB, dB, rowsB, gsB, isB)

        @pl.when(j % 2 == 1)
        def _():
            _step(j, sB, dB, rowsB, gsB, isB, sA, dA, rowsA, gsA, isA)

        return carry

    lax.fori_loop(0, cnt, body, 0)
    plsc.subcore_barrier()
    pltpu.sync_copy(acc.at[pl.ds(s * rpt, rpt)], out_hbm.at[c, pl.ds(s * rpt, rpt)])


_prop_call = pl.kernel(
    _prop_body,
    out_type=jax.ShapeDtypeStruct((NC, TROWS, 128), jnp.float32),
    mesh=_mesh,
    scratch_types=[
        pltpu.VMEM_SHARED((TROWS, 128), jnp.float32),
        pltpu.VMEM((CHUNK,), jnp.int32),
        pltpu.VMEM((CHUNK,), jnp.int32),
        pltpu.VMEM((CHUNK,), jnp.int32),
        pltpu.VMEM((CHUNK,), jnp.int32),
        pltpu.VMEM((CHUNK, 128), jnp.float32),
        pltpu.VMEM((CHUNK, 128), jnp.float32),
        pltpu.SemaphoreType.DMA,
        pltpu.SemaphoreType.DMA,
        pltpu.SemaphoreType.DMA,
        pltpu.SemaphoreType.DMA,
    ],
)


def _tc1_body(x_ref, w_ref, deg_ref, z_ref, dinv_ref):
    a = deg_ref[...]
    deg = jnp.max(a[0], axis=-1) + jnp.max(a[1], axis=-1) + 1.0
    dv = lax.rsqrt(deg)
    dinv_ref[...] = dv
    z_ref[...] = dv[:, None] * jnp.dot(x_ref[...], w_ref[...],
                                       preferred_element_type=jnp.float32)


def _tc1(x, w1, degp):
    return pl.pallas_call(
        _tc1_body,
        grid=(NBLK,),
        in_specs=[
            pl.BlockSpec((NB, N_FEAT), lambda i: (i, 0)),
            pl.BlockSpec((N_FEAT, HIDDEN), lambda i: (0, 0)),
            pl.BlockSpec((NC, NB, 128), lambda i: (0, i, 0)),
        ],
        out_specs=[
            pl.BlockSpec((NB, HIDDEN), lambda i: (i, 0)),
            pl.BlockSpec((NB,), lambda i: (i,)),
        ],
        out_shape=[
            jax.ShapeDtypeStruct((TROWS, HIDDEN), jnp.float32),
            jax.ShapeDtypeStruct((N_NODES,), jnp.float32),
        ],
    )(x, w1, degp)


def _tc2_body(s_ref, dinv_ref, b_ref, w_ref, z_ref):
    a = s_ref[...]
    f = a[0] + a[1]
    dv = dinv_ref[...][:, None]
    f = jnp.maximum(dv * f + b_ref[...][None, :], 0.0)
    z_ref[...] = dv * jnp.dot(f, w_ref[...], preferred_element_type=jnp.float32)


def _tc2(sp, dinv, b, w):
    return pl.pallas_call(
        _tc2_body,
        grid=(NBLK,),
        in_specs=[
            pl.BlockSpec((NC, NB, HIDDEN), lambda i: (0, i, 0)),
            pl.BlockSpec((NB,), lambda i: (i,)),
            pl.BlockSpec((HIDDEN,), lambda i: (0,)),
            pl.BlockSpec((HIDDEN, HIDDEN), lambda i: (0, 0)),
        ],
        out_specs=pl.BlockSpec((NB, HIDDEN), lambda i: (i, 0)),
        out_shape=jax.ShapeDtypeStruct((TROWS, HIDDEN), jnp.float32),
    )(sp, dinv, b, w)


def _tcf_body(s_ref, dinv_ref, b_ref, batch_ref, wl_ref, bl_ref, wo_ref,
              bo_ref, out_ref, sums_ref, cnt_ref):
    i = pl.program_id(0)
    nblocks = pl.num_programs(0)

    @pl.when(i == 0)
    def _():
        sums_ref[...] = jnp.zeros_like(sums_ref)
        cnt_ref[...] = jnp.zeros_like(cnt_ref)

    a = s_ref[...]
    f = a[0] + a[1]
    dv = dinv_ref[...][:, None]
    f = jnp.maximum(dv * f + b_ref[...][None, :], 0.0)
    rowf = i * NB + lax.broadcasted_iota(jnp.int32, (NB, HIDDEN), 0)
    f = jnp.where(rowf < N_NODES, f, 0.0)

    ids = batch_ref[...][:, None]
    gid = lax.broadcasted_iota(jnp.int32, (NB, N_GRAPHS), 1)
    row = i * NB + lax.broadcasted_iota(jnp.int32, (NB, N_GRAPHS), 0)
    m = ((ids == gid) & (row < N_NODES)).astype(jnp.float32)
    sums_ref[...] += lax.dot_general(m, f, (((0,), (0,)), ((), ())),
                                     preferred_element_type=jnp.float32)
    cnt_ref[...] += jnp.sum(m, axis=0)

    @pl.when(i == nblocks - 1)
    def _():
        pooled = sums_ref[...] / jnp.clip(cnt_ref[...], 1.0, None)[:, None]
        g = jnp.maximum(
            jnp.dot(pooled, wl_ref[...], preferred_element_type=jnp.float32)
            + bl_ref[...][None, :], 0.0)
        out_ref[...] = (jnp.dot(g, wo_ref[...],
                                preferred_element_type=jnp.float32)
                        + bo_ref[...][None, :])


def _tcf(sp, dinv, b, batch, wl, bl, wo, bo):
    return pl.pallas_call(
        _tcf_body,
        grid=(NBLK,),
        in_specs=[
            pl.BlockSpec((NC, NB, HIDDEN), lambda i: (0, i, 0)),
            pl.BlockSpec((NB,), lambda i: (i,)),
            pl.BlockSpec((HIDDEN,), lambda i: (0,)),
            pl.BlockSpec((NB,), lambda i: (i,)),
            pl.BlockSpec((HIDDEN, HIDDEN // 2), lambda i: (0, 0)),
            pl.BlockSpec((HIDDEN // 2,), lambda i: (0,)),
            pl.BlockSpec((HIDDEN // 2, 1), lambda i: (0, 0)),
            pl.BlockSpec((1,), lambda i: (0,)),
        ],
        out_specs=pl.BlockSpec((N_GRAPHS, 1), lambda i: (0, 0)),
        out_shape=jax.ShapeDtypeStruct((N_GRAPHS, 1), jnp.float32),
        scratch_shapes=[
            pltpu.VMEM((N_GRAPHS, HIDDEN), jnp.float32),
            pltpu.VMEM((N_GRAPHS,), jnp.float32),
        ],
    )(sp, dinv, b, batch, wl, bl, wo, bo)


@jax.jit
def _run(x, edge_index, batch, W1, b1, W2, b2, W3, b3, Wl, bl, Wo, bo):
    src = edge_index[0].astype(jnp.int32)
    dst = edge_index[1].astype(jnp.int32)
    pad = E_PAD - N_EDGES
    fill = DUMP + jnp.arange(pad, dtype=jnp.int32) % (TROWS - N_NODES)
    src_p = jnp.concatenate([src, fill])
    dst_p = jnp.concatenate([dst, fill])
    src_c = src_p.reshape(NC, NS, ECH, CHUNK)
    dst_c = dst_p.reshape(NC, NS, ECH, CHUNK)
    src_t = src_p.reshape(NCHT, CHUNK)
    dst_t = dst_p.reshape(NCHT, CHUNK)
    zeros = jnp.zeros((TROWS, 128), jnp.float32)
    ones = jnp.ones((CHUNK, 128), jnp.float32)
    batch_i = batch.astype(jnp.int32)

    degp = _deg_call(zeros, ones, dst_c)
    z1, dinv = _tc1(x, W1, degp)
    s1 = _prop_call(z1, zeros, src_t, dst_t)
    z2 = _tc2(s1, dinv, b1, W2)
    s2 = _prop_call(z2, zeros, src_t, dst_t)
    z3 = _tc2(s2, dinv, b2, W3)
    s3 = _prop_call(z3, zeros, src_t, dst_t)
    return _tcf(s3, dinv, b3, batch_i, Wl, bl, Wo, bo)


def kernel(x, edge_index, batch, W1, b1, W2, b2, W3, b3, Wl, bl, Wo, bo):
    return _run(x, edge_index, batch, W1, b1, W2, b2, W3, b3, Wl, bl, Wo, bo)


# speedup vs baseline: 1.3260x; 1.1253x over previous
import jax
import jax.numpy as jnp
from jax import lax
from jax.experimental import pallas as pl
from jax.experimental.pallas import tpu as pltpu
from jax.experimental.pallas import tpu_sc as plsc

N_NODES = 10000
N_FEAT = 128
HIDDEN = 128
N_GRAPHS = 512
N_EDGES = 320000

NC = 2
NS = 16
CHUNK = 128
E_PAD = 327680
ECH = E_PAD // NC // NS // CHUNK
NCHT = E_PAD // CHUNK
PCH0 = NCHT // 2
PCH1 = NCHT - PCH0
DUMP = N_NODES
TROWS = 10112
NB = 512
NBLK = 20

_mesh = plsc.VectorSubcoreMesh(
    core_axis_name="c", subcore_axis_name="s", num_cores=NC, num_subcores=NS)


def _deg_body(zeros_hbm, ones_hbm, dst_hbm, out_hbm, acc, ones_v, dA, dB,
              isA, isB):
    c = lax.axis_index("c")
    s = lax.axis_index("s")
    rpt = TROWS // NS
    pltpu.sync_copy(zeros_hbm.at[pl.ds(s * rpt, rpt)], acc.at[pl.ds(s * rpt, rpt)])

    pltpu.sync_copy(ones_hbm, ones_v)
    plsc.subcore_barrier()

    pltpu.sync_copy(dst_hbm.at[c, s, 0], dA)
    pltpu.async_copy(dst_hbm.at[c, s, 1], dB, isB)

    def _step(j, dC, isC, dN, isN):
        @pl.when(j + 1 < ECH)
        def _():
            pltpu.make_async_copy(dst_hbm.at[c, s, j + 1], dN, isN).wait()

        pltpu.sync_copy(ones_v, acc.at[dC], add=True)

        @pl.when(j + 2 < ECH)
        def _():
            pltpu.async_copy(dst_hbm.at[c, s, j + 2], dC, isC)

    def body(j, carry):
        @pl.when(j % 2 == 0)
        def _():
            _step(j, dA, isA, dB, isB)

        @pl.when(j % 2 == 1)
        def _():
            _step(j, dB, isB, dA, isA)

        return carry

    lax.fori_loop(0, ECH, body, 0)
    plsc.subcore_barrier()
    pltpu.sync_copy(acc.at[pl.ds(s * rpt, rpt)], out_hbm.at[c, pl.ds(s * rpt, rpt)])


_deg_call = pl.kernel(
    _deg_body,
    out_type=jax.ShapeDtypeStruct((NC, TROWS, 128), jnp.float32),
    mesh=_mesh,
    scratch_types=[
        pltpu.VMEM_SHARED((TROWS, 128), jnp.float32),
        pltpu.VMEM((CHUNK, 128), jnp.float32),
        pltpu.VMEM((CHUNK,), jnp.int32),
        pltpu.VMEM((CHUNK,), jnp.int32),
        pltpu.SemaphoreType.DMA,
        pltpu.SemaphoreType.DMA,
    ],
)


def _prop_body(z_hbm, zeros_hbm, src_hbm, dst_hbm, out_hbm, acc,
               sA, dA, sB, dB, rowsA, rowsB, gsA, gsB, isA, isB):
    c = lax.axis_index("c")
    s = lax.axis_index("s")
    rpt = TROWS // NS
    @pl.when(c == 0)
    def _():
        pltpu.sync_copy(z_hbm.at[pl.ds(s * rpt, rpt)], acc.at[pl.ds(s * rpt, rpt)])

    @pl.when(c != 0)
    def _():
        pltpu.sync_copy(zeros_hbm.at[pl.ds(s * rpt, rpt)],
                        acc.at[pl.ds(s * rpt, rpt)])

    plsc.subcore_barrier()

    n0 = PCH0 // NS
    n1 = PCH1 // NS
    cnt = jnp.where(c == 0, n0, n1)
    base = jnp.where(c == 0, s * n0, PCH0 + s * n1)

    pltpu.sync_copy(src_hbm.at[base], sA)
    pltpu.sync_copy(dst_hbm.at[base], dA)
    pltpu.async_copy(z_hbm.at[sA], rowsA, gsA)
    pltpu.async_copy(src_hbm.at[base + 1], sB, isB)
    pltpu.async_copy(dst_hbm.at[base + 1], dB, isB)

    def _step(j, sC, dC, rowsC, gsC, isC, sN, dN, rowsN, gsN, isN):
        @pl.when(j + 1 < cnt)
        def _():
            pltpu.make_async_copy(src_hbm.at[base + j + 1], sN, isN).wait()
            pltpu.make_async_copy(dst_hbm.at[base + j + 1], dN, isN).wait()
            pltpu.async_copy(z_hbm.at[sN], rowsN, gsN)

        pltpu.make_async_copy(z_hbm.at[sC], rowsC, gsC).wait()
        pltpu.sync_copy(rowsC, acc.at[dC], add=True)

        @pl.when(j + 2 < cnt)
        def _():
            pltpu.async_copy(src_hbm.at[base + j + 2], sC, isC)
            pltpu.async_copy(dst_hbm.at[base + j + 2], dC, isC)

    def body(j, carry):
        @pl.when(j % 2 == 0)
        def _():
            _step(j, sA, dA, rowsA, gsA, isA, sB, dB, rowsB, gsB, isB)

        @pl.when(j % 2 == 1)
        def _():
            _step(j, sB, dB, rowsB, gsB, isB, sA, dA, rowsA, gsA, isA)

        return carry

    lax.fori_loop(0, cnt, body, 0)
    plsc.subcore_barrier()
    pltpu.sync_copy(acc.at[pl.ds(s * rpt, rpt)], out_hbm.at[c, pl.ds(s * rpt, rpt)])


_prop_call = pl.kernel(
    _prop_body,
    out_type=jax.ShapeDtypeStruct((NC, TROWS, 128), jnp.float32),
    mesh=_mesh,
    scratch_types=[
        pltpu.VMEM_SHARED((TROWS, 128), jnp.float32),
        pltpu.VMEM((CHUNK,), jnp.int32),
        pltpu.VMEM((CHUNK,), jnp.int32),
        pltpu.VMEM((CHUNK,), jnp.int32),
        pltpu.VMEM((CHUNK,), jnp.int32),
        pltpu.VMEM((CHUNK, 128), jnp.float32),
        pltpu.VMEM((CHUNK, 128), jnp.float32),
        pltpu.SemaphoreType.DMA,
        pltpu.SemaphoreType.DMA,
        pltpu.SemaphoreType.DMA,
        pltpu.SemaphoreType.DMA,
    ],
)


def _tc1_body(x_ref, w_ref, deg_ref, z_ref, dinv_ref):
    a = deg_ref[...]
    deg = jnp.max(a[0], axis=-1) + jnp.max(a[1], axis=-1) + 1.0
    dv = lax.rsqrt(deg)
    dinv_ref[...] = dv
    z_ref[...] = dv[:, None] * jnp.dot(x_ref[...], w_ref[...],
                                       preferred_element_type=jnp.float32)


def _tc1(x, w1, degp):
    return pl.pallas_call(
        _tc1_body,
        grid=(NBLK,),
        in_specs=[
            pl.BlockSpec((NB, N_FEAT), lambda i: (i, 0)),
            pl.BlockSpec((N_FEAT, HIDDEN), lambda i: (0, 0)),
            pl.BlockSpec((NC, NB, 128), lambda i: (0, i, 0)),
        ],
        out_specs=[
            pl.BlockSpec((NB, HIDDEN), lambda i: (i, 0)),
            pl.BlockSpec((NB,), lambda i: (i,)),
        ],
        out_shape=[
            jax.ShapeDtypeStruct((TROWS, HIDDEN), jnp.float32),
            jax.ShapeDtypeStruct((N_NODES,), jnp.float32),
        ],
    )(x, w1, degp)


def _tc2_body(s_ref, dinv_ref, b_ref, w_ref, z_ref):
    a = s_ref[...]
    f = a[0] + a[1]
    dv = dinv_ref[...][:, None]
    f = jnp.maximum(dv * f + b_ref[...][None, :], 0.0)
    z_ref[...] = dv * jnp.dot(f, w_ref[...], preferred_element_type=jnp.float32)


def _tc2(sp, dinv, b, w):
    return pl.pallas_call(
        _tc2_body,
        grid=(NBLK,),
        in_specs=[
            pl.BlockSpec((NC, NB, HIDDEN), lambda i: (0, i, 0)),
            pl.BlockSpec((NB,), lambda i: (i,)),
            pl.BlockSpec((HIDDEN,), lambda i: (0,)),
            pl.BlockSpec((HIDDEN, HIDDEN), lambda i: (0, 0)),
        ],
        out_specs=pl.BlockSpec((NB, HIDDEN), lambda i: (i, 0)),
        out_shape=jax.ShapeDtypeStruct((TROWS, HIDDEN), jnp.float32),
    )(sp, dinv, b, w)


def _tcf_body(s_ref, dinv_ref, b_ref, batch_ref, wl_ref, bl_ref, wo_ref,
              bo_ref, out_ref, sums_ref, cnt_ref):
    i = pl.program_id(0)
    nblocks = pl.num_programs(0)

    @pl.when(i == 0)
    def _():
        sums_ref[...] = jnp.zeros_like(sums_ref)
        cnt_ref[...] = jnp.zeros_like(cnt_ref)

    a = s_ref[...]
    f = a[0] + a[1]
    dv = dinv_ref[...][:, None]
    f = jnp.maximum(dv * f + b_ref[...][None, :], 0.0)
    rowf = i * NB + lax.broadcasted_iota(jnp.int32, (NB, HIDDEN), 0)
    f = jnp.where(rowf < N_NODES, f, 0.0)

    ids = batch_ref[...][:, None]
    gid = lax.broadcasted_iota(jnp.int32, (NB, N_GRAPHS), 1)
    row = i * NB + lax.broadcasted_iota(jnp.int32, (NB, N_GRAPHS), 0)
    m = ((ids == gid) & (row < N_NODES)).astype(jnp.float32)
    sums_ref[...] += lax.dot_general(m, f, (((0,), (0,)), ((), ())),
                                     preferred_element_type=jnp.float32)
    cnt_ref[...] += jnp.sum(m, axis=0)

    @pl.when(i == nblocks - 1)
    def _():
        pooled = sums_ref[...] / jnp.clip(cnt_ref[...], 1.0, None)[:, None]
        g = jnp.maximum(
            jnp.dot(pooled, wl_ref[...], preferred_element_type=jnp.float32)
            + bl_ref[...][None, :], 0.0)
        out_ref[...] = (jnp.dot(g, wo_ref[...],
                                preferred_element_type=jnp.float32)
                        + bo_ref[...][None, :])


def _tcf(sp, dinv, b, batch, wl, bl, wo, bo):
    return pl.pallas_call(
        _tcf_body,
        grid=(NBLK,),
        in_specs=[
            pl.BlockSpec((NC, NB, HIDDEN), lambda i: (0, i, 0)),
            pl.BlockSpec((NB,), lambda i: (i,)),
            pl.BlockSpec((HIDDEN,), lambda i: (0,)),
            pl.BlockSpec((NB,), lambda i: (i,)),
            pl.BlockSpec((HIDDEN, HIDDEN // 2), lambda i: (0, 0)),
            pl.BlockSpec((HIDDEN // 2,), lambda i: (0,)),
            pl.BlockSpec((HIDDEN // 2, 1), lambda i: (0, 0)),
            pl.BlockSpec((1,), lambda i: (0,)),
        ],
        out_specs=pl.BlockSpec((N_GRAPHS, 1), lambda i: (0, 0)),
        out_shape=jax.ShapeDtypeStruct((N_GRAPHS, 1), jnp.float32),
        scratch_shapes=[
            pltpu.VMEM((N_GRAPHS, HIDDEN), jnp.float32),
            pltpu.VMEM((N_GRAPHS,), jnp.float32),
        ],
    )(sp, dinv, b, batch, wl, bl, wo, bo)


@jax.jit
def _run(x, edge_index, batch, W1, b1, W2, b2, W3, b3, Wl, bl, Wo, bo):
    src = edge_index[0].astype(jnp.int32)
    dst = edge_index[1].astype(jnp.int32)
    pad = E_PAD - N_EDGES
    fill = DUMP + jnp.arange(pad, dtype=jnp.int32) % (TROWS - N_NODES)
    src_p = jnp.concatenate([src, fill])
    dst_p = jnp.concatenate([dst, fill])
    src_c = src_p.reshape(NC, NS, ECH, CHUNK)
    dst_c = dst_p.reshape(NC, NS, ECH, CHUNK)
    src_t = src_p.reshape(NCHT, CHUNK)
    dst_t = dst_p.reshape(NCHT, CHUNK)
    zeros = jnp.zeros((TROWS, 128), jnp.float32)
    ones = jnp.ones((CHUNK, 128), jnp.float32)
    batch_i = batch.astype(jnp.int32)

    degp = _deg_call(zeros, ones, dst_c)
    z1, dinv = _tc1(x, W1, degp)
    s1 = _prop_call(z1, zeros, src_t, dst_t)
    z2 = _tc2(s1, dinv, b1, W2)
    s2 = _prop_call(z2, zeros, src_t, dst_t)
    z3 = _tc2(s2, dinv, b2, W3)
    s3 = _prop_call(z3, zeros, src_t, dst_t)
    return _tcf(s3, dinv, b3, batch_i, Wl, bl, Wo, bo)


def kernel(x, edge_index, batch, W1, b1, W2, b2, W3, b3, Wl, bl, Wo, bo):
    return _run(x, edge_index, batch, W1, b1, W2, b2, W3, b3, Wl, bl, Wo, bo)
